# Initial kernel scaffold; baseline (speedup 1.0000x reference)
import jax
import jax.numpy as jnp
from jax.experimental import pallas as pl


def kernel(coords, tables):
    raise NotImplementedError("write your pallas kernel here")


# speedup vs baseline: 33.8783x; 33.8783x over previous
import functools

import jax
import jax.numpy as jnp
import numpy as np
from jax import lax
from jax.experimental import pallas as pl
from jax.experimental.pallas import tpu as pltpu
from jax.experimental.pallas import tpu_sc as plsc

NUM_LODS = 16
FEATURE_DIM = 2
TABLE_SIZE = 2 ** 19
MIN_RES = 16
MAX_RES = 512
P1 = 2654435761
P2 = 805459861
MASK = TABLE_SIZE - 1

L = 16
NW = 32
CHUNK = 128
NGROUP = CHUNK // L
BLK = 16


def _resolutions():
    b = np.exp((np.log(MAX_RES) - np.log(MIN_RES)) / (NUM_LODS - 1))
    return [int(np.floor(MIN_RES * (b ** l))) for l in range(NUM_LODS)]


_RES_BCAST = np.broadcast_to(
    np.array(_resolutions(), np.float32)[:, None], (NUM_LODS, L)).copy()
_LOFS_BCAST = np.broadcast_to(
    (np.arange(NUM_LODS, dtype=np.int32) * TABLE_SIZE)[:, None],
    (NUM_LODS, L)).copy()

_OFFS = [(i, j, k) for i in (0, 1) for j in (0, 1) for k in (0, 1)]


def _encoder_body(n_points, coords_hbm, tables_hbm, resb_hbm, lofs_hbm,
                  out_hbm, cc, wb, wofs, acc, resv, lofsv, sem, *cbufs):
    idxb = cbufs[0:8]
    rows = cbufs[8:16]
    per_worker = n_points // NW
    n_chunks = per_worker // CHUNK
    wid = lax.axis_index("s") * 2 + lax.axis_index("c")
    base = wid * per_worker

    pltpu.sync_copy(resb_hbm, resv)
    pltpu.sync_copy(lofs_hbm, lofsv)

    iota = lax.iota(jnp.int32, L)
    fiota = iota.astype(jnp.float32) * 0.0
    zero_i = iota * 0
    onef = fiota + 1.0

    def chunk_body(ch, _):
        cbase = base + ch * CHUNK
        pltpu.sync_copy(coords_hbm.at[pl.ds(cbase * 3, CHUNK * 3)], cc)

        def level_body(lvl, _):
            res = resv[lvl, :]
            lofs = lofsv[lvl, :]

            def idx_group(g, _):
                p0 = g * L
                t = (p0 + iota) * 3
                xs = plsc.load_gather(cc, [t])
                ys = plsc.load_gather(cc, [t + 1])
                zs = plsc.load_gather(cc, [t + 2])
                fx = xs * res
                fy = ys * res
                fz = zs * res
                px = fx.astype(jnp.int32)
                py = fy.astype(jnp.int32)
                pz = fz.astype(jnp.int32)
                rx = fx - px.astype(jnp.float32)
                ry = fy - py.astype(jnp.float32)
                rz = fz - pz.astype(jnp.float32)
                a0 = px.astype(jnp.uint32)
                a1 = py.astype(jnp.uint32) * jnp.uint32(P1)
                a2 = pz.astype(jnp.uint32) * jnp.uint32(P2)
                b0 = a0 + jnp.uint32(1)
                b1 = a1 + jnp.uint32(P1)
                b2 = a2 + jnp.uint32(P2)
                wx = (onef - rx, rx)
                wy = (onef - ry, ry)
                wz = (onef - rz, rz)
                ha = (a0, b0)
                hb = (a1, b1)
                hc = (a2, b2)
                for ci, (i, j, k) in enumerate(_OFFS):
                    h = ha[i] ^ hb[j] ^ hc[k]
                    row = (h & jnp.uint32(MASK)).astype(jnp.int32) + lofs
                    idxb[ci][pl.ds(p0, L)] = lax.shift_right_logical(row, 3)
                    wofs[ci, pl.ds(p0, L)] = lax.shift_left(row & 7, 1)
                    wb[ci, pl.ds(p0, L)] = wx[i] * wy[j] * wz[k]
                return 0

            lax.fori_loop(0, NGROUP, idx_group, 0)

            descs = [
                pltpu.async_copy(tables_hbm.at[idxb[ci]], rows[ci], sem)
                for ci in range(8)
            ]
            for d in descs:
                d.wait()

            lvl2 = lax.shift_right_logical(lofs, 18)

            def comb_group(g, _):
                p0 = g * L
                pv = p0 + iota
                acc0 = fiota
                acc1 = fiota
                for ci in range(8):
                    lov = wofs[ci, pl.ds(p0, L)]
                    f0 = plsc.load_gather(rows[ci], [pv, lov])
                    f1 = plsc.load_gather(rows[ci], [pv, lov + 1])
                    w = wb[ci, pl.ds(p0, L)]
                    acc0 = acc0 + w * f0
                    acc1 = acc1 + w * f1
                oidx = pv * (2 * NUM_LODS) + lvl2
                plsc.store_scatter(acc, [oidx], acc0)
                plsc.store_scatter(acc, [oidx + 1], acc1)
                return 0

            lax.fori_loop(0, NGROUP, comb_group, 0)
            return 0

        lax.fori_loop(0, NUM_LODS, level_body, 0)
        pltpu.sync_copy(
            acc, out_hbm.at[pl.ds(cbase * (2 * NUM_LODS), CHUNK * 2 * NUM_LODS)])
        return 0

    lax.fori_loop(0, n_chunks, chunk_body, 0)


def kernel(coords, tables):
    batch, num_samples, _ = coords.shape
    n = batch * num_samples
    coords_flat = coords.reshape(n * 3)
    n_blocks = NUM_LODS * TABLE_SIZE * FEATURE_DIM // BLK
    tables_blk = tables.reshape(n_blocks, BLK)
    resb = jnp.asarray(_RES_BCAST)
    lofs = jnp.asarray(_LOFS_BCAST)

    mesh = plsc.VectorSubcoreMesh(core_axis_name="c", subcore_axis_name="s")
    run = pl.kernel(
        functools.partial(_encoder_body, n),
        out_type=jax.ShapeDtypeStruct((n * 2 * NUM_LODS,), jnp.float32),
        mesh=mesh,
        compiler_params=pltpu.CompilerParams(
            needs_layout_passes=False, use_tc_tiling_on_sc=False),
        scratch_types=[
            pltpu.VMEM((CHUNK * 3,), jnp.float32),
            pltpu.VMEM((8, CHUNK), jnp.float32),
            pltpu.VMEM((8, CHUNK), jnp.int32),
            pltpu.VMEM((CHUNK * 2 * NUM_LODS,), jnp.float32),
            pltpu.VMEM((NUM_LODS, L), jnp.float32),
            pltpu.VMEM((NUM_LODS, L), jnp.int32),
            pltpu.SemaphoreType.DMA,
        ] + [pltpu.VMEM((CHUNK,), jnp.int32) for _ in range(8)]
          + [pltpu.VMEM((CHUNK, BLK), jnp.float32) for _ in range(8)],
    )
    out = run(coords_flat, tables_blk, resb, lofs)
    return out.reshape(batch, num_samples, 2 * NUM_LODS)
